# baseline (speedup 1.0000x reference)
import functools

import numpy as np
import jax
import jax.numpy as jnp
from jax import lax
from jax.experimental import pallas as pl
from jax.experimental.pallas import tpu as pltpu
from jax.experimental.pallas import tpu_sc as plsc

WINDOW_SIZE = 100000
E = 64
B = 4096
S = 200
SCALE = 8.0

NC = 2
NS = 16
NW = NC * NS
BT = B // NW
ET = E // 8
L = 16
NBUF = 4


def _positional_encoding() -> np.ndarray:
    half = E // 2
    positions = np.arange(S, dtype=np.float32)[:, None]
    depths = np.arange(half, dtype=np.float32)[None, :] / float(half)
    angle_rads = positions * (1.0 / (10000.0 ** depths))
    return np.concatenate(
        [np.sin(angle_rads), np.cos(angle_rads)], axis=-1
    ).astype(np.float32)


_POSB = np.repeat(_positional_encoding()[:, :, None], L, axis=2)


_MESH = plsc.VectorSubcoreMesh(core_axis_name="c", subcore_axis_name="s")


@functools.partial(
    pl.kernel,
    mesh=_MESH,
    compiler_params=pltpu.CompilerParams(
        use_tc_tiling_on_sc=False, needs_layout_passes=False
    ),
    out_type=jax.ShapeDtypeStruct((S, ET, NW, 8, BT), jnp.float32),
    scratch_types=[
        pltpu.VMEM((BT, S), jnp.int32),
    ]
    + [pltpu.VMEM((BT,), jnp.int32) for _ in range(NBUF)]
    + [pltpu.VMEM((BT, E), jnp.float32) for _ in range(NBUF)]
    + [pltpu.VMEM((E, L), jnp.float32) for _ in range(NBUF)]
    + [pltpu.VMEM((ET, 8, BT), jnp.float32) for _ in range(NBUF)]
    + [pltpu.SemaphoreType.DMA for _ in range(2 * NBUF)],
)
def _embed_pos(x_hbm, table_hbm, posb_hbm, out_hbm, idx_v, *bufs_sems):
    sidx = bufs_sems[:NBUF]
    gbufs = bufs_sems[NBUF : 2 * NBUF]
    pbufs = bufs_sems[2 * NBUF : 3 * NBUF]
    obufs = bufs_sems[3 * NBUF : 4 * NBUF]
    gsems = bufs_sems[4 * NBUF : 5 * NBUF]
    ssems = bufs_sems[5 * NBUF :]
    wid = lax.axis_index("s") * NC + lax.axis_index("c")
    seq0 = wid * BT
    pltpu.sync_copy(x_hbm.at[pl.ds(seq0, BT)], idx_v)

    def gather_start(s, b):
        lanes = lax.iota(jnp.int32, L)
        col = jnp.broadcast_to(s, (L,)).astype(jnp.int32)
        for bc in range(BT // L):
            sidx[b][pl.ds(bc * L, L)] = plsc.load_gather(
                idx_v, [bc * L + lanes, col]
            )
        pltpu.async_copy(table_hbm.at[sidx[b]], gbufs[b], gsems[b])
        pltpu.async_copy(posb_hbm.at[s], pbufs[b], gsems[b])

    def compute(s, b):
        gbuf, pbuf, obuf = gbufs[b], pbufs[b], obufs[b]

        @plsc.parallel_loop(0, E, unroll=8)
        def e_body(e):
            p = pbuf[e, :]
            lanes = lax.iota(jnp.int32, L)
            ecol = jnp.broadcast_to(e, (L,)).astype(jnp.int32)
            for bc in range(BT // L):
                r = plsc.load_gather(gbuf, [bc * L + lanes, ecol])
                obuf[e // 8, e % 8, pl.ds(bc * L, L)] = r * SCALE + p

    def store_start(s, b):
        pltpu.async_copy(obufs[b], out_hbm.at[s, :, wid], ssems[b])

    def store_wait(b):
        pltpu.make_async_copy(
            obufs[b], out_hbm.at[0, :, 0], ssems[b]
        ).wait()

    def gather_wait(b):
        pltpu.make_async_copy(
            table_hbm.at[sidx[b]], gbufs[b], gsems[b]
        ).wait()
        pltpu.make_async_copy(posb_hbm.at[0], pbufs[b], gsems[b]).wait()

    for b in range(NBUF):
        gather_start(b, b)

    def s_loop(s0, carry):
        for b in range(NBUF):
            s = s0 * NBUF + b
            gather_wait(b)
            @pl.when(s >= NBUF)
            def _():
                store_wait(b)

            compute(s, b)
            store_start(s, b)

            @pl.when(s + NBUF < S)
            def _():
                gather_start(s + NBUF, b)

        return carry

    lax.fori_loop(0, S // NBUF, s_loop, 0)
    for b in range(NBUF):
        store_wait(b)


@jax.jit
def _run(x, table):
    out5 = _embed_pos(x.astype(jnp.int32), table, jnp.asarray(_POSB))
    return out5.transpose(2, 4, 0, 1, 3).reshape(B, S, E)


def kernel(x, table):
    return _run(x, table)


# speedup vs baseline: 1.0921x; 1.0921x over previous
import functools

import numpy as np
import jax
import jax.numpy as jnp
from jax import lax
from jax.experimental import pallas as pl
from jax.experimental.pallas import tpu as pltpu
from jax.experimental.pallas import tpu_sc as plsc

WINDOW_SIZE = 100000
E = 64
B = 4096
S = 200
SCALE = 8.0

NC = 2
NS = 16
NW = NC * NS
BT = B // NW
ET = E // 8
L = 16
NBUF = 4


def _positional_encoding() -> np.ndarray:
    half = E // 2
    positions = np.arange(S, dtype=np.float32)[:, None]
    depths = np.arange(half, dtype=np.float32)[None, :] / float(half)
    angle_rads = positions * (1.0 / (10000.0 ** depths))
    return np.concatenate(
        [np.sin(angle_rads), np.cos(angle_rads)], axis=-1
    ).astype(np.float32)


_POSB = np.repeat(_positional_encoding()[:, :, None], L, axis=2)


_MESH = plsc.VectorSubcoreMesh(core_axis_name="c", subcore_axis_name="s")


@functools.partial(
    pl.kernel,
    mesh=_MESH,
    compiler_params=pltpu.CompilerParams(
        use_tc_tiling_on_sc=False, needs_layout_passes=False
    ),
    out_type=jax.ShapeDtypeStruct((S, ET, NW, 8, BT), jnp.float32),
    scratch_types=[
        pltpu.VMEM((BT, S), jnp.int32),
    ]
    + [pltpu.VMEM((BT,), jnp.int32) for _ in range(NBUF)]
    + [pltpu.VMEM((BT, E), jnp.float32) for _ in range(NBUF)]
    + [pltpu.VMEM((E, L), jnp.float32) for _ in range(NBUF)]
    + [pltpu.VMEM((E, BT), jnp.float32) for _ in range(NBUF)]
    + [pltpu.SemaphoreType.DMA for _ in range(2 * NBUF)],
)
def _embed_pos(x_hbm, table_hbm, posb_hbm, out_hbm, idx_v, *bufs_sems):
    sidx = bufs_sems[:NBUF]
    gbufs = bufs_sems[NBUF : 2 * NBUF]
    pbufs = bufs_sems[2 * NBUF : 3 * NBUF]
    obufs = bufs_sems[3 * NBUF : 4 * NBUF]
    gsems = bufs_sems[4 * NBUF : 5 * NBUF]
    ssems = bufs_sems[5 * NBUF :]
    wid = lax.axis_index("s") * NC + lax.axis_index("c")
    seq0 = wid * BT
    pltpu.sync_copy(x_hbm.at[pl.ds(seq0, BT)], idx_v)

    def gather_start(s, b):
        lanes = lax.iota(jnp.int32, L)
        col = jnp.broadcast_to(s, (L,)).astype(jnp.int32)
        for bc in range(BT // L):
            sidx[b][pl.ds(bc * L, L)] = plsc.load_gather(
                idx_v, [bc * L + lanes, col]
            )
        pltpu.async_copy(table_hbm.at[sidx[b]], gbufs[b], gsems[b])
        pltpu.async_copy(posb_hbm.at[s], pbufs[b], gsems[b])

    def compute(s, b):
        gbuf, pbuf, obuf = gbufs[b], pbufs[b], obufs[b]

        @plsc.parallel_loop(0, E, unroll=4)
        def e_body(e):
            p = pbuf[e, :]
            lanes = lax.iota(jnp.int32, L)
            ecol = jnp.broadcast_to(e, (L,)).astype(jnp.int32)
            for bc in range(BT // L):
                r = plsc.load_gather(gbuf, [bc * L + lanes, ecol])
                obuf[e, pl.ds(bc * L, L)] = r * SCALE + p

    def store_start(s, b):
        for te in range(ET):
            pltpu.async_copy(
                obufs[b].at[pl.ds(te * 8, 8)],
                out_hbm.at[s, te, wid],
                ssems[b],
            )

    def store_wait(b):
        for te in range(ET):
            pltpu.make_async_copy(
                obufs[b].at[pl.ds(te * 8, 8)],
                out_hbm.at[0, te, 0],
                ssems[b],
            ).wait()

    def gather_wait(b):
        pltpu.make_async_copy(
            table_hbm.at[sidx[b]], gbufs[b], gsems[b]
        ).wait()
        pltpu.make_async_copy(posb_hbm.at[0], pbufs[b], gsems[b]).wait()

    for b in range(NBUF):
        gather_start(b, b)

    def s_loop(s0, carry):
        for b in range(NBUF):
            s = s0 * NBUF + b
            gather_wait(b)
            @pl.when(s >= NBUF)
            def _():
                store_wait(b)

            compute(s, b)
            store_start(s, b)

            @pl.when(s + NBUF < S)
            def _():
                gather_start(s + NBUF, b)

        return carry

    lax.fori_loop(0, S // NBUF, s_loop, 0)
    for b in range(NBUF):
        store_wait(b)


@jax.jit
def _run(x, table):
    out5 = _embed_pos(x.astype(jnp.int32), table, jnp.asarray(_POSB))
    return out5.transpose(2, 4, 0, 1, 3).reshape(B, S, E)


def kernel(x, table):
    return _run(x, table)
